# baseline (speedup 1.0000x reference)
import jax
import jax.numpy as jnp
from jax import lax
from jax.experimental import pallas as pl
from jax.experimental.pallas import tpu as pltpu
from jax.experimental.pallas import tpu_sc as plsc

NC = 2
NS = 16
NW = NC * NS

BATCH = 4096
SEQ = 200
D = 64
HALF = SEQ // 2
SEQS_PER_W = BATCH // NW


def _body(idx_hbm, table_hbm, pos_hbm, out_hbm, idx_v, pos_v, rows_v, sem):
    c = lax.axis_index("c")
    s = lax.axis_index("s")
    wid = s * NC + c

    pltpu.sync_copy(idx_hbm.at[wid], idx_v)
    pltpu.sync_copy(pos_hbm.at[pl.ds(0, SEQ)], pos_v)

    def seq_step(g, carry):
        cp0 = pltpu.async_copy(
            table_hbm.at[idx_v.at[2 * g]], rows_v.at[pl.ds(0, HALF)], sem)
        cp1 = pltpu.async_copy(
            table_hbm.at[idx_v.at[2 * g + 1]], rows_v.at[pl.ds(HALF, HALF)], sem)
        cp0.wait()
        cp1.wait()

        def row(r, carry2):
            for cc in range(D // 16):
                sl = pl.ds(cc * 16, 16)
                rows_v[r, sl] = rows_v[r, sl] + pos_v[r, sl]
            return carry2

        lax.fori_loop(0, SEQ, row, 0, unroll=2)
        pltpu.sync_copy(rows_v, out_hbm.at[wid * SEQS_PER_W + g])
        return carry

    lax.fori_loop(0, SEQS_PER_W, seq_step, 0)


@jax.jit
def _embed(x3d, word_table, pos_table):
    mesh = plsc.VectorSubcoreMesh(
        core_axis_name="c", subcore_axis_name="s", num_cores=NC, num_subcores=NS
    )
    kfn = pl.kernel(
        _body,
        out_type=jax.ShapeDtypeStruct((BATCH, SEQ, D), jnp.float32),
        mesh=mesh,
        compiler_params=pltpu.CompilerParams(use_tc_tiling_on_sc=False),
        scratch_types=[
            pltpu.VMEM((2 * SEQS_PER_W, HALF), jnp.int32),
            pltpu.VMEM((SEQ, D), jnp.float32),
            pltpu.VMEM((SEQ, D), jnp.float32),
            pltpu.SemaphoreType.DMA,
        ],
    )
    return kfn(x3d, word_table, pos_table)


def kernel(x, word_table, pos_table):
    x3d = x.astype(jnp.int32).reshape(NW, 2 * SEQS_PER_W, HALF)
    return _embed(x3d, word_table, pos_table)
